# baseline (speedup 1.0000x reference)
import numpy as np

import jax
import jax.numpy as jnp
from jax.experimental import pallas as pl
from jax.experimental.pallas import tpu as pltpu
from jax.sharding import Mesh, PartitionSpec as P

N_NODES = 2048
N_IN = 512
N_OUT = 256

BK = 128
NB = (N_NODES - N_IN) // BK
NSER = (N_NODES - N_OUT - N_IN) // BK
GRP = 8
NGRP = BK // GRP
KC = 256
NKC = N_NODES // KC


def _dag_kernel(xT_ref, w_ref, wd_ref, b_ref, yT_ref, aT_ref, z_ref):
    t = pl.program_id(0)
    batch = xT_ref.shape[1]

    @pl.when(t == 0)
    def _init():
        aT_ref[0:N_IN, :] = xT_ref[...]
        aT_ref[N_IN:, :] = jnp.zeros((N_NODES - N_IN, batch), jnp.float32)

    s = N_IN + t * BK
    kmax = jnp.minimum(s + BK, N_NODES - N_OUT)

    z_ref[...] = (
        jnp.dot(w_ref[:, 0:3 * KC], aT_ref[0:3 * KC, :],
                preferred_element_type=jnp.float32)
        + b_ref[0]
    )
    for c in range(3, NKC):
        @pl.when(KC * c < kmax)
        def _chunk(c=c):
            z_ref[...] += jnp.dot(w_ref[:, KC * c:KC * (c + 1)],
                                  aT_ref[KC * c:KC * (c + 1), :],
                                  preferred_element_type=jnp.float32)

    @pl.when(t < NSER)
    def _serial():
        zw = z_ref[0:GRP, :]
        for g in range(NGRP):
            lo = g * GRP
            last = g == NGRP - 1
            if not last:
                zw_next = z_ref[lo + GRP:lo + 2 * GRP, :]
                if g > 0:
                    zw_next += jnp.dot(
                        wd_ref[lo + GRP:lo + 2 * GRP, 0:lo],
                        aT_ref[pl.ds(s, lo), :],
                        preferred_element_type=jnp.float32)
                wd_win = wd_ref[lo:lo + 2 * GRP, lo:lo + GRP]
            else:
                wd_win = wd_ref[lo:lo + GRP, lo:lo + GRP]
            rows = []
            for i in range(GRP):
                arow = jnp.tanh(zw[i:i + 1, :])
                rows.append(arow)
                zw = zw + wd_win[0:GRP, i:i + 1] * arow
                if not last:
                    zw_next = zw_next + wd_win[GRP:2 * GRP, i:i + 1] * arow
            a8 = jnp.concatenate(rows, axis=0)
            aT_ref[pl.ds(s + lo, GRP), :] = a8
            if not last:
                zw = zw_next

    @pl.when(t >= NSER)
    def _emit_output():
        yT_ref[...] = jnp.tanh(z_ref[...])


def _forward(x, W, b):
    batch = x.shape[0]
    xT = x.T
    b3 = b[N_IN:].reshape(NB, BK, 1)
    hb = N_IN // BK

    yT = pl.pallas_call(
        _dag_kernel,
        grid=(NB,),
        in_specs=[
            pl.BlockSpec((N_IN, batch), lambda t: (0, 0)),
            pl.BlockSpec((BK, N_NODES), lambda t: (hb + t, 0)),
            pl.BlockSpec((BK, BK), lambda t: (hb + t, hb + t)),
            pl.BlockSpec((1, BK, 1), lambda t: (t, 0, 0)),
        ],
        out_specs=pl.BlockSpec((BK, batch), lambda t: (jnp.maximum(t - NSER, 0), 0)),
        out_shape=jax.ShapeDtypeStruct((N_OUT, batch), jnp.float32),
        scratch_shapes=[
            pltpu.VMEM((N_NODES, batch), jnp.float32),
            pltpu.VMEM((BK, batch), jnp.float32),
        ],
    )(xT, W, W, b3)
    return yT.T


def kernel(x, W, b):
    devs = jax.devices()
    n = len(devs)
    while n > 1 and x.shape[0] % n:
        n -= 1
    if n <= 1:
        return _forward(x, W, b)
    mesh = Mesh(np.array(devs[:n]), ("d",))
    f = jax.shard_map(_forward, mesh=mesh,
                      in_specs=(P("d", None), P(None, None), P(None)),
                      out_specs=P("d", None), check_vma=False)
    return f(x, W, b)
